# baseline (speedup 1.0000x reference)
import functools

import jax
import jax.numpy as jnp
import numpy as np
from jax import lax
from jax.experimental import pallas as pl
from jax.experimental.pallas import tpu as pltpu
from jax.experimental.pallas import tpu_sc as plsc

N = 10000
E = 160000
D = 256
NUM_CONVS = 5
KS = [8000, 6400, 5120, 4096, 3277]
MB = 400

_SHARD_SIZES = ([5040] * 11 + [4928] * 4 + [4848]) * 2
_STARTS = np.concatenate([[0], np.cumsum(_SHARD_SIZES)]).astype(np.int32)
NW = 32
NCH = 48
CH = 112
TR = N + 112
RPW = TR // 16
TRASH = TR - 1


def _mp_sc_call(featA, featB, islab, zslice):
    mesh = plsc.VectorSubcoreMesh(core_axis_name="c", subcore_axis_name="s")

    @functools.partial(
        pl.kernel,
        mesh=mesh,
        out_type=[jax.ShapeDtypeStruct((2, TR, 128), jnp.float32),
                  jax.ShapeDtypeStruct((2, TR, 128), jnp.float32)],
        scratch_types=[
            pltpu.VMEM((3, CH), jnp.int32),
            pltpu.VMEM((CH, 128), jnp.float32),
            pltpu.VMEM((CH, 128), jnp.float32),
            pltpu.VMEM_SHARED((TR, 128), jnp.float32),
            pltpu.SemaphoreType.DMA,
        ],
    )
    def k(featA_h, featB_h, islab_h, z_h, outA_h, outB_h,
          ib, st, rb, tab, semG):
        c = lax.axis_index("c")
        s = lax.axis_index("s")
        w = c * 16 + s
        zero16 = jnp.zeros((16,), jnp.float32)
        for feat_h, out_h in ((featA_h, outA_h), (featB_h, outB_h)):
            pltpu.sync_copy(z_h, tab.at[pl.ds(s * RPW, RPW)])
            plsc.subcore_barrier()

            def body(ci, accs):
                pltpu.sync_copy(islab_h.at[w, ci], ib)
                pltpu.async_copy(feat_h.at[ib.at[0]], st, semG).wait()
                new = list(accs)
                for r in range(CH):
                    if r % 16 == 0:
                        fl16 = ib[1, pl.ds(r, 16)]
                    keep = fl16[jnp.full((16,), r % 16, jnp.int32)
                                ].astype(jnp.float32)
                    for g in range(8):
                        row = st[r, pl.ds(g * 16, 16)]
                        a = row + new[g] * keep
                        new[g] = a
                        rb[r, pl.ds(g * 16, 16)] = a
                pltpu.sync_copy(rb, tab.at[ib.at[2]], add=True)
                return tuple(new)

            lax.fori_loop(0, NCH, body, tuple([zero16] * 8))
            plsc.subcore_barrier()
            pltpu.sync_copy(tab.at[pl.ds(s * RPW, RPW)],
                            out_h.at[c, pl.ds(s * RPW, RPW)])
            plsc.subcore_barrier()

    return k(featA, featB, islab, zslice)


def _edge_plan(src, dst):
    order = jnp.argsort(dst, stable=True)
    osrc = src[order]
    odst = dst[order]
    starts = jnp.asarray(_STARTS[:-1])
    pos = (starts[:, None, None]
           + (jnp.arange(NCH, dtype=jnp.int32) * CH)[None, :, None]
           + jnp.arange(CH, dtype=jnp.int32)[None, None, :])
    ends = jnp.asarray(_STARTS[1:])
    valid = pos < ends[:, None, None]
    posc = jnp.minimum(pos, E - 1)
    src_slab = jnp.where(valid, osrc[posc], 0).astype(jnp.int32)
    pdst = odst[posc]
    run_start = jnp.searchsorted(odst, pdst, side="left")
    run_end = jnp.searchsorted(odst, pdst, side="right") - 1
    flag_slab = 1 - ((run_start == pos) | (pos == starts[:, None, None])
                     | ~valid).astype(jnp.int32)
    is_end = (run_end == pos) | (pos == ends[:, None, None] - 1)
    divert = run_start <= starts[:, None, None]
    row = jnp.where(divert, N + jnp.arange(NW, dtype=jnp.int32)[:, None, None], pdst)
    rowend_slab = jnp.where(valid & is_end, row, TRASH).astype(jnp.int32)
    first_dst = odst[starts]
    islab = jnp.stack([src_slab, flag_slab, rowend_slab], axis=2)
    return islab, first_dst, odst, osrc


def _seg_sum_exact(table, islab, zslice, first_dst):
    tA, tB = _mp_sc_call(table[:, :128], table[:, 128:], islab, zslice)
    sA = tA[0] + tA[1]
    sB = tB[0] + tB[1]
    aggA = sA[:N].at[first_dst].add(sA[N:N + NW])
    aggB = sB[:N].at[first_dst].add(sB[N:N + NW])
    return jnp.concatenate([aggA, aggB], axis=1)


def _mm_kernel(a_ref, w_ref, b_ref, o_ref, *, relu):
    acc = lax.dot_general(a_ref[...], w_ref[...], (((1,), (0,)), ((), ())),
                          precision="default", preferred_element_type=jnp.float32)
    acc = acc + b_ref[...]
    if relu:
        acc = jnp.maximum(acc, 0.0)
    o_ref[...] = acc


def _matmul(a, w, b, relu):
    m, k = a.shape
    n = w.shape[1]
    grid = (m // MB,)
    return pl.pallas_call(
        functools.partial(_mm_kernel, relu=relu),
        grid=grid,
        in_specs=[
            pl.BlockSpec((MB, k), lambda i: (i, 0)),
            pl.BlockSpec((k, n), lambda i: (0, 0)),
            pl.BlockSpec((1, n), lambda i: (0, 0)),
        ],
        out_specs=pl.BlockSpec((MB, n), lambda i: (i, 0)),
        out_shape=jax.ShapeDtypeStruct((m, n), jnp.float32),
    )(a, w, b)


def _readout_kernel(f_ref, m_ref, o_ref):
    i = pl.program_id(0)
    f = f_ref[...]
    msk = m_ref[...]
    bsum = jnp.sum(f, axis=0, keepdims=True)
    bmax = jnp.max(jnp.where(msk > 0, f, -1e30), axis=0, keepdims=True)

    @pl.when(i == 0)
    def _():
        o_ref[:, :D] = bsum
        o_ref[:, D:] = bmax

    @pl.when(i > 0)
    def _():
        o_ref[:, :D] = o_ref[:, :D] + bsum
        o_ref[:, D:] = jnp.maximum(o_ref[:, D:], bmax)


def _readout(feat, mask2d):
    return pl.pallas_call(
        _readout_kernel,
        grid=(N // MB,),
        in_specs=[
            pl.BlockSpec((MB, D), lambda i: (i, 0)),
            pl.BlockSpec((MB, D), lambda i: (i, 0)),
        ],
        out_specs=pl.BlockSpec((1, 2 * D), lambda i: (0, 0)),
        out_shape=jax.ShapeDtypeStruct((1, 2 * D), jnp.float32),
    )(feat, mask2d)


def _mlp_layer_kernel(a_ref, w_ref, b_ref, o_ref):
    acc = lax.dot_general(a_ref[...], w_ref[...], (((1,), (0,)), ((), ())),
                          precision="default", preferred_element_type=jnp.float32)
    acc = acc + b_ref[...]
    o_ref[...] = jnp.clip(acc, -10.0, 10.0)


def _mlp_layer(a, w, b):
    k = a.shape[1]
    n = w.shape[1]
    del k
    return pl.pallas_call(
        _mlp_layer_kernel,
        out_shape=jax.ShapeDtypeStruct((8, n), jnp.float32),
    )(a, w, b)


def _pad_cols(x, n):
    return jnp.pad(x, ((0, 0), (0, n - x.shape[1])))


def kernel(x, edge_index, Wc, bc, Ws, bs, Wl0, bl0, Wl1, bl1, Wl2, bl2,
           Wl3, bl3, Wl4, bl4, Wl5, bl5):
    src = edge_index[0]
    dst = edge_index[1]
    islab, first_dst, odst, osrc = _edge_plan(src, dst)
    order2 = jnp.argsort(src, stable=True)
    ssrc = src[order2]
    sdst2 = dst[order2]
    zslice = jnp.zeros((RPW, 128), jnp.float32)

    node_mask = jnp.ones((N,), dtype=x.dtype)
    feat = x
    readouts = []
    perms = []
    for i in range(NUM_CONVS):
        deg_in = jax.ops.segment_sum(node_mask[osrc] * node_mask[odst], odst,
                                     num_segments=N, indices_are_sorted=True)
        deg_out = jax.ops.segment_sum(node_mask[ssrc] * node_mask[sdst2], ssrc,
                                      num_segments=N, indices_are_sorted=True)
        norm_out = 1.0 / jnp.sqrt(jnp.maximum(deg_out, 1.0))
        norm_in = 1.0 / jnp.sqrt(jnp.maximum(deg_in, 1.0))

        hpre = feat * norm_out[:, None]
        agg = _seg_sum_exact(hpre, islab, zslice, first_dst)
        agg = agg * norm_in[:, None]
        h = _matmul(agg, Wc[i], bc[i][None, :], relu=True)

        hs = h * norm_out[:, None] * node_mask[:, None]
        agg2 = _seg_sum_exact(hs, islab, zslice, first_dst)
        agg2 = agg2 * norm_in[:, None]
        score = _matmul(agg2, _pad_cols(Ws[i], 128), _pad_cols(bs[i][None, :], 128),
                        relu=False)[:, 0]

        masked_score = jnp.where(node_mask > 0, score, -1e30)
        _, perm = lax.top_k(masked_score, KS[i])
        new_mask = jnp.zeros((N,), dtype=x.dtype).at[perm].set(1.0)
        feat = h * jnp.tanh(score)[:, None] * new_mask[:, None]
        node_mask = new_mask

        ro = _readout(feat, jnp.broadcast_to(new_mask[:, None], (N, D)))
        readouts.append(jnp.concatenate([ro[:, :D] / KS[i], ro[:, D:]], axis=-1))
        perms.append(perm)

    fr = jnp.concatenate(readouts, axis=-1)
    fr = jnp.clip(fr, -10.0, 10.0)
    fr = jnp.pad(fr, ((0, 7), (0, 128 - fr.shape[1] % 128 if fr.shape[1] % 128 else 0)))
    dims_pad = [2688, 1792, 1152, 768, 512, 128]
    layers = [(Wl0, bl0), (Wl1, bl1), (Wl2, bl2), (Wl3, bl3), (Wl4, bl4), (Wl5, bl5)]
    a = fr
    for (w, b), np_ in zip(layers, dims_pad):
        wp = jnp.pad(w, ((0, a.shape[1] - w.shape[0]), (0, np_ - w.shape[1])))
        bp = _pad_cols(b[None, :], np_)
        a = _mlp_layer(a, wp, bp)
    val = jnp.clip(a[0:1, 0], -10.0, 10.0) * 100.0
    return val, jnp.concatenate(perms).astype(jnp.float32)


# speedup vs baseline: 1.1521x; 1.1521x over previous
import functools

import jax
import jax.numpy as jnp
import numpy as np
from jax import lax
from jax.experimental import pallas as pl
from jax.experimental.pallas import tpu as pltpu
from jax.experimental.pallas import tpu_sc as plsc

N = 10000
E = 160000
D = 256
NUM_CONVS = 5
KS = [8000, 6400, 5120, 4096, 3277]
MB = 400

_SHARD_SIZES = ([5040] * 11 + [4928] * 4 + [4848]) * 2
_STARTS = np.concatenate([[0], np.cumsum(_SHARD_SIZES)]).astype(np.int32)
NW = 32
NCH = 45
CH = 112
TR = N + 112
RPW = TR // 16
TRASH = TR - 1


def _mp_sc_call(featA, featB, islab, fslab, zslice):
    mesh = plsc.VectorSubcoreMesh(core_axis_name="c", subcore_axis_name="s")

    @functools.partial(
        pl.kernel,
        mesh=mesh,
        out_type=[jax.ShapeDtypeStruct((2, TR, 128), jnp.float32),
                  jax.ShapeDtypeStruct((2, TR, 128), jnp.float32)],
        scratch_types=[
            pltpu.VMEM((2, CH), jnp.int32),
            pltpu.VMEM((CH,), jnp.float32),
            pltpu.VMEM((CH, 128), jnp.float32),
            pltpu.VMEM((CH, 128), jnp.float32),
            pltpu.VMEM_SHARED((TR, 128), jnp.float32),
            pltpu.SemaphoreType.DMA,
        ],
    )
    def k(featA_h, featB_h, islab_h, fslab_h, z_h, outA_h, outB_h,
          ib, flags_row, st, rb, tab, semG):
        c = lax.axis_index("c")
        s = lax.axis_index("s")
        w = c * 16 + s
        zero16 = jnp.zeros((16,), jnp.float32)
        for feat_h, out_h in ((featA_h, outA_h), (featB_h, outB_h)):
            pltpu.sync_copy(z_h, tab.at[pl.ds(s * RPW, RPW)])
            plsc.subcore_barrier()

            def body(ci, accs):
                pltpu.sync_copy(islab_h.at[w, ci], ib)
                pltpu.sync_copy(fslab_h.at[w, ci], flags_row)
                pltpu.async_copy(feat_h.at[ib.at[0]], st, semG).wait()
                new = list(accs)
                for r in range(CH):
                    if r % 16 == 0:
                        fl16 = flags_row[pl.ds(r, 16)]
                    keep = fl16[jnp.full((16,), r % 16, jnp.int32)]
                    for g in range(8):
                        row = st[r, pl.ds(g * 16, 16)]
                        a = row + new[g] * keep
                        new[g] = a
                        rb[r, pl.ds(g * 16, 16)] = a
                pltpu.sync_copy(rb, tab.at[ib.at[1]], add=True)
                return tuple(new)

            lax.fori_loop(0, NCH, body, tuple([zero16] * 8))
            plsc.subcore_barrier()
            pltpu.sync_copy(tab.at[pl.ds(s * RPW, RPW)],
                            out_h.at[c, pl.ds(s * RPW, RPW)])
            plsc.subcore_barrier()

    return k(featA, featB, islab, fslab, zslice)


def _edge_plan(src, dst):
    order = jnp.argsort(dst, stable=True)
    osrc = src[order]
    odst = dst[order]
    starts = jnp.asarray(_STARTS[:-1])
    pos = (starts[:, None, None]
           + (jnp.arange(NCH, dtype=jnp.int32) * CH)[None, :, None]
           + jnp.arange(CH, dtype=jnp.int32)[None, None, :])
    ends = jnp.asarray(_STARTS[1:])
    valid = pos < ends[:, None, None]
    posc = jnp.minimum(pos, E - 1)
    src_slab = jnp.where(valid, osrc[posc], 0).astype(jnp.int32)
    pdst = odst[posc]
    run_start = jnp.searchsorted(odst, pdst, side="left")
    run_end = jnp.searchsorted(odst, pdst, side="right") - 1
    flag_slab = 1.0 - ((run_start == pos) | (pos == starts[:, None, None])
                       | ~valid).astype(jnp.float32)
    is_end = (run_end == pos) | (pos == ends[:, None, None] - 1)
    divert = run_start <= starts[:, None, None]
    row = jnp.where(divert, N + jnp.arange(NW, dtype=jnp.int32)[:, None, None], pdst)
    rowend_slab = jnp.where(valid & is_end, row, TRASH).astype(jnp.int32)
    first_dst = odst[starts]
    islab = jnp.stack([src_slab, rowend_slab], axis=2)
    return islab, flag_slab, first_dst, odst, osrc


def _seg_sum_exact(table, islab, fslab, zslice, first_dst):
    tA, tB = _mp_sc_call(table[:, :128], table[:, 128:], islab, fslab, zslice)
    sA = tA[0] + tA[1]
    sB = tB[0] + tB[1]
    aggA = sA[:N].at[first_dst].add(sA[N:N + NW])
    aggB = sB[:N].at[first_dst].add(sB[N:N + NW])
    return jnp.concatenate([aggA, aggB], axis=1)


def _mm_kernel(a_ref, w_ref, b_ref, o_ref, *, relu):
    acc = lax.dot_general(a_ref[...], w_ref[...], (((1,), (0,)), ((), ())),
                          precision="default", preferred_element_type=jnp.float32)
    acc = acc + b_ref[...]
    if relu:
        acc = jnp.maximum(acc, 0.0)
    o_ref[...] = acc


def _matmul(a, w, b, relu):
    m, k = a.shape
    n = w.shape[1]
    grid = (m // MB,)
    return pl.pallas_call(
        functools.partial(_mm_kernel, relu=relu),
        grid=grid,
        in_specs=[
            pl.BlockSpec((MB, k), lambda i: (i, 0)),
            pl.BlockSpec((k, n), lambda i: (0, 0)),
            pl.BlockSpec((1, n), lambda i: (0, 0)),
        ],
        out_specs=pl.BlockSpec((MB, n), lambda i: (i, 0)),
        out_shape=jax.ShapeDtypeStruct((m, n), jnp.float32),
    )(a, w, b)


def _readout_kernel(f_ref, m_ref, o_ref):
    i = pl.program_id(0)
    f = f_ref[...]
    msk = m_ref[...]
    bsum = jnp.sum(f, axis=0, keepdims=True)
    bmax = jnp.max(jnp.where(msk > 0, f, -1e30), axis=0, keepdims=True)

    @pl.when(i == 0)
    def _():
        o_ref[:, :D] = bsum
        o_ref[:, D:] = bmax

    @pl.when(i > 0)
    def _():
        o_ref[:, :D] = o_ref[:, :D] + bsum
        o_ref[:, D:] = jnp.maximum(o_ref[:, D:], bmax)


def _readout(feat, mask2d):
    return pl.pallas_call(
        _readout_kernel,
        grid=(N // MB,),
        in_specs=[
            pl.BlockSpec((MB, D), lambda i: (i, 0)),
            pl.BlockSpec((MB, D), lambda i: (i, 0)),
        ],
        out_specs=pl.BlockSpec((1, 2 * D), lambda i: (0, 0)),
        out_shape=jax.ShapeDtypeStruct((1, 2 * D), jnp.float32),
    )(feat, mask2d)


def _mlp_layer_kernel(a_ref, w_ref, b_ref, o_ref):
    acc = lax.dot_general(a_ref[...], w_ref[...], (((1,), (0,)), ((), ())),
                          precision="default", preferred_element_type=jnp.float32)
    acc = acc + b_ref[...]
    o_ref[...] = jnp.clip(acc, -10.0, 10.0)


def _mlp_layer(a, w, b):
    k = a.shape[1]
    n = w.shape[1]
    del k
    return pl.pallas_call(
        _mlp_layer_kernel,
        out_shape=jax.ShapeDtypeStruct((8, n), jnp.float32),
    )(a, w, b)


def _pad_cols(x, n):
    return jnp.pad(x, ((0, 0), (0, n - x.shape[1])))


def kernel(x, edge_index, Wc, bc, Ws, bs, Wl0, bl0, Wl1, bl1, Wl2, bl2,
           Wl3, bl3, Wl4, bl4, Wl5, bl5):
    src = edge_index[0]
    dst = edge_index[1]
    islab, fslab, first_dst, odst, osrc = _edge_plan(src, dst)
    order2 = jnp.argsort(src, stable=True)
    ssrc = src[order2]
    sdst2 = dst[order2]
    zslice = jnp.zeros((RPW, 128), jnp.float32)

    node_mask = jnp.ones((N,), dtype=x.dtype)
    feat = x
    readouts = []
    perms = []
    for i in range(NUM_CONVS):
        deg_in = jax.ops.segment_sum(node_mask[osrc] * node_mask[odst], odst,
                                     num_segments=N, indices_are_sorted=True)
        deg_out = jax.ops.segment_sum(node_mask[ssrc] * node_mask[sdst2], ssrc,
                                      num_segments=N, indices_are_sorted=True)
        norm_out = 1.0 / jnp.sqrt(jnp.maximum(deg_out, 1.0))
        norm_in = 1.0 / jnp.sqrt(jnp.maximum(deg_in, 1.0))

        hpre = feat * norm_out[:, None]
        agg = _seg_sum_exact(hpre, islab, fslab, zslice, first_dst)
        agg = agg * norm_in[:, None]
        h = _matmul(agg, Wc[i], bc[i][None, :], relu=True)

        hs = h * norm_out[:, None] * node_mask[:, None]
        agg2 = _seg_sum_exact(hs, islab, fslab, zslice, first_dst)
        agg2 = agg2 * norm_in[:, None]
        score = _matmul(agg2, _pad_cols(Ws[i], 128), _pad_cols(bs[i][None, :], 128),
                        relu=False)[:, 0]

        masked_score = jnp.where(node_mask > 0, score, -1e30)
        _, perm = lax.top_k(masked_score, KS[i])
        new_mask = jnp.zeros((N,), dtype=x.dtype).at[perm].set(1.0)
        feat = h * jnp.tanh(score)[:, None] * new_mask[:, None]
        node_mask = new_mask

        ro = _readout(feat, jnp.broadcast_to(new_mask[:, None], (N, D)))
        readouts.append(jnp.concatenate([ro[:, :D] / KS[i], ro[:, D:]], axis=-1))
        perms.append(perm)

    fr = jnp.concatenate(readouts, axis=-1)
    fr = jnp.clip(fr, -10.0, 10.0)
    fr = jnp.pad(fr, ((0, 7), (0, 128 - fr.shape[1] % 128 if fr.shape[1] % 128 else 0)))
    dims_pad = [2688, 1792, 1152, 768, 512, 128]
    layers = [(Wl0, bl0), (Wl1, bl1), (Wl2, bl2), (Wl3, bl3), (Wl4, bl4), (Wl5, bl5)]
    a = fr
    for (w, b), np_ in zip(layers, dims_pad):
        wp = jnp.pad(w, ((0, a.shape[1] - w.shape[0]), (0, np_ - w.shape[1])))
        bp = _pad_cols(b[None, :], np_)
        a = _mlp_layer(a, wp, bp)
    val = jnp.clip(a[0:1, 0], -10.0, 10.0) * 100.0
    return val, jnp.concatenate(perms).astype(jnp.float32)
